# baseline (speedup 1.0000x reference)
import functools

import jax
import jax.numpy as jnp
import numpy as np
from jax import lax
from jax.experimental import pallas as pl
from jax.experimental.pallas import tpu as pltpu
from jax.experimental.pallas import tpu_sc as plsc

N = 4096
E = 65536
HID = 16
NC = 10
NUM_GRAPHS = 16
NUM_CLASSES = 10
SC_CORES = 2
SC_SUBCORES = 16
NWORK = SC_CORES * SC_SUBCORES
EPW = E // NWORK
CH = 128
NCHUNK = EPW // CH
ROWS_PT = N // SC_SUBCORES

_mesh = plsc.VectorSubcoreMesh(
    core_axis_name="c", subcore_axis_name="s", num_cores=SC_CORES)
_SC_PARAMS = pltpu.CompilerParams(use_tc_tiling_on_sc=False)


@functools.partial(
    pl.kernel,
    out_type=(jax.ShapeDtypeStruct((E, 8), jnp.float32),
              jax.ShapeDtypeStruct((E, 8), jnp.float32)),
    mesh=_mesh,
    compiler_params=_SC_PARAMS,
    scratch_types=(pltpu.VMEM((CH,), jnp.int32),
                   pltpu.VMEM((CH,), jnp.int32),
                   pltpu.VMEM((CH, 8), jnp.float32),
                   pltpu.VMEM((CH, 8), jnp.float32),
                   pltpu.SemaphoreType.DMA,
                   pltpu.SemaphoreType.DMA),
)
def _sc_gather_records(p_hbm, row_hbm, col_hbm, gr_hbm, gc_hbm,
                       idxr_v, idxc_v, rr_v, rc_v, sem1, sem2):
    wid = lax.axis_index("s") * SC_CORES + lax.axis_index("c")
    base = wid * EPW

    @pl.loop(0, NCHUNK)
    def _chunk(i):
        off = base + i * CH
        pltpu.sync_copy(row_hbm.at[pl.ds(off, CH)], idxr_v)
        pltpu.sync_copy(col_hbm.at[pl.ds(off, CH)], idxc_v)
        cp1 = pltpu.async_copy(p_hbm.at[idxr_v], rr_v, sem1)
        cp2 = pltpu.async_copy(p_hbm.at[idxc_v], rc_v, sem2)
        cp1.wait()
        cp2.wait()
        pltpu.sync_copy(rr_v, gr_hbm.at[pl.ds(off, CH)])
        pltpu.sync_copy(rc_v, gc_hbm.at[pl.ds(off, CH)])


SW2 = 32


@functools.partial(
    pl.kernel,
    out_type=jax.ShapeDtypeStruct((SC_CORES, N, SW2), jnp.float32),
    mesh=_mesh,
    compiler_params=_SC_PARAMS,
    scratch_types=(pltpu.VMEM((CH,), jnp.int32),
                   pltpu.VMEM((CH, SW2), jnp.float32),
                   pltpu.VMEM_SHARED((N, SW2), jnp.float32)),
)
def _sc_scatter_m(m_hbm, col_hbm, zeros_hbm, out_hbm, idx_v, m_v, acc_sh):
    cid = lax.axis_index("c")
    sid = lax.axis_index("s")
    pltpu.sync_copy(zeros_hbm.at[pl.ds(sid * ROWS_PT, ROWS_PT)],
                    acc_sh.at[pl.ds(sid * ROWS_PT, ROWS_PT)])
    plsc.subcore_barrier()
    base = cid * (E // SC_CORES) + sid * EPW
    @pl.loop(0, NCHUNK)
    def _chunk(i):
        off = base + i * CH
        pltpu.sync_copy(col_hbm.at[pl.ds(off, CH)], idx_v)
        pltpu.sync_copy(m_hbm.at[pl.ds(off, CH)], m_v)
        pltpu.sync_copy(m_v, acc_sh.at[idx_v], add=True)
    plsc.subcore_barrier()
    pltpu.sync_copy(acc_sh.at[pl.ds(sid * ROWS_PT, ROWS_PT)],
                    out_hbm.at[cid, pl.ds(sid * ROWS_PT, ROWS_PT)])


@functools.partial(
    pl.kernel,
    out_type=jax.ShapeDtypeStruct((SC_CORES, N, HID), jnp.float32),
    mesh=_mesh,
    compiler_params=_SC_PARAMS,
    scratch_types=(pltpu.VMEM((CH,), jnp.int32),
                   pltpu.VMEM((CH,), jnp.int32),
                   pltpu.VMEM((CH, HID), jnp.float32),
                   pltpu.VMEM_SHARED((N, HID), jnp.float32),
                   pltpu.SemaphoreType.DMA),
)
def _sc_gather_scatter(hn_hbm, row_hbm, col_hbm, zeros_hbm, out_hbm,
                       idxr_v, idxc_v, rows_v, acc_sh, sem):
    cid = lax.axis_index("c")
    sid = lax.axis_index("s")
    pltpu.sync_copy(zeros_hbm.at[pl.ds(sid * ROWS_PT, ROWS_PT)],
                    acc_sh.at[pl.ds(sid * ROWS_PT, ROWS_PT)])
    plsc.subcore_barrier()
    base = cid * (E // SC_CORES) + sid * EPW
    @pl.loop(0, NCHUNK)
    def _chunk(i):
        off = base + i * CH
        pltpu.sync_copy(row_hbm.at[pl.ds(off, CH)], idxr_v)
        pltpu.sync_copy(col_hbm.at[pl.ds(off, CH)], idxc_v)
        pltpu.async_copy(hn_hbm.at[idxr_v], rows_v, sem).wait()
        pltpu.sync_copy(rows_v, acc_sh.at[idxc_v], add=True)
    plsc.subcore_barrier()
    pltpu.sync_copy(acc_sh.at[pl.ds(sid * ROWS_PT, ROWS_PT)],
                    out_hbm.at[cid, pl.ds(sid * ROWS_PT, ROWS_PT)])


_DOT = dict(precision=lax.Precision.HIGHEST, preferred_element_type=jnp.float32)
BE = 2048


def _tc_edge_body(gr_ref, gc_ref, w24_ref, thr160_ref, thr16_ref, wc_ref,
                  m_ref):
    gr = gr_ref[...]
    gc = gc_ref[...]
    cat = jnp.concatenate([gr, gc], axis=1)
    out = jnp.dot(cat, w24_ref[...], **_DOT)
    relu1 = jnp.maximum(out, 0.0)
    prod = gr * gc
    d2pix = (gr[:, 5:6] + gc[:, 5:6]
             - 2.0 * jnp.sum(prod[:, 0:3], axis=1, keepdims=True))
    d2pos = (gr[:, 6:7] + gc[:, 6:7]
             - 2.0 * jnp.sum(prod[:, 3:5], axis=1, keepdims=True))
    dpix = jnp.sqrt(jnp.maximum(d2pix, 0.0))
    dpos = jnp.sqrt(jnp.maximum(d2pos, 0.0))
    ev = (jnp.exp(dpos * (-1.0 / ((0.05 * np.pi) ** 2)))
          + jnp.exp(dpix * (-1.0 / (0.05 ** 2))))
    m160 = jnp.where(ev >= thr160_ref[...], relu1, 0.0)
    mask16 = jnp.where(ev >= thr16_ref[...], 1.0, 0.0)
    cat2 = jnp.concatenate([m160, mask16], axis=1)
    v = jnp.dot(cat2, wc_ref[...], **_DOT)
    m_ref[...] = jnp.concatenate(
        [v, jnp.ones((BE, 1), jnp.float32), jnp.zeros((BE, 15), jnp.float32)],
        axis=1)


def _tc_edge(gr, gc, w24, thr160_row, thr16_row, wc):
    grid = E // BE
    return pl.pallas_call(
        _tc_edge_body,
        grid=(grid,),
        in_specs=[
            pl.BlockSpec((BE, 8), lambda i: (i, 0)),
            pl.BlockSpec((BE, 8), lambda i: (i, 0)),
            pl.BlockSpec((16, NC * HID), lambda i: (0, 0)),
            pl.BlockSpec((1, NC * HID), lambda i: (0, 0)),
            pl.BlockSpec((1, 16), lambda i: (0, 0)),
            pl.BlockSpec((176, HID), lambda i: (0, 0)),
        ],
        out_specs=pl.BlockSpec((BE, SW2), lambda i: (i, 0)),
        out_shape=jax.ShapeDtypeStruct((E, SW2), jnp.float32),
        compiler_params=pltpu.CompilerParams(
            dimension_semantics=("arbitrary",)),
    )(gr, gc, w24, thr160_row, thr16_row, wc)


def _tc_gcn1_body(x_ref, s_ref, wx_ref, hn_ref, dinv_ref):
    s = s_ref[0] + s_ref[1]
    deg = 1.0 + s[:, HID:HID + 1]
    dinv = lax.rsqrt(deg)
    h0 = jnp.dot(x_ref[...], wx_ref[...], **_DOT) + s[:, 0:HID]
    hn_ref[...] = h0 * dinv
    dinv_ref[...] = jnp.broadcast_to(dinv, (N, HID))


def _tc_gcn1(x, s, wx3):
    return pl.pallas_call(
        _tc_gcn1_body,
        in_specs=[pl.BlockSpec((N, 3), lambda: (0, 0)),
                  pl.BlockSpec((SC_CORES, N, SW2), lambda: (0, 0, 0)),
                  pl.BlockSpec((3, HID), lambda: (0, 0))],
        out_specs=[pl.BlockSpec((N, HID), lambda: (0, 0)),
                   pl.BlockSpec((N, HID), lambda: (0, 0))],
        out_shape=[jax.ShapeDtypeStruct((N, HID), jnp.float32),
                   jax.ShapeDtypeStruct((N, HID), jnp.float32)],
    )(x, s, wx3)


def _tc_gcn2_body(t2_ref, hn_ref, dinv_ref, b1_ref, w2_ref, hn2_ref):
    t = t2_ref[0] + t2_ref[1]
    g1 = jnp.maximum(dinv_ref[...] * (t + hn_ref[...]) + b1_ref[...], 0.0)
    hn2_ref[...] = jnp.dot(g1, w2_ref[...], **_DOT) * dinv_ref[...]


def _tc_gcn2(t2, hn, dinv, b1row, w2):
    return pl.pallas_call(
        _tc_gcn2_body,
        in_specs=[pl.BlockSpec((SC_CORES, N, HID), lambda: (0, 0, 0)),
                  pl.BlockSpec((N, HID), lambda: (0, 0)),
                  pl.BlockSpec((N, HID), lambda: (0, 0)),
                  pl.BlockSpec((1, HID), lambda: (0, 0)),
                  pl.BlockSpec((HID, HID), lambda: (0, 0))],
        out_specs=pl.BlockSpec((N, HID), lambda: (0, 0)),
        out_shape=jax.ShapeDtypeStruct((N, HID), jnp.float32),
    )(t2, hn, dinv, b1row, w2)


def _tc_final_body(t2_ref, hn2_ref, dinv_ref, b2_ref, batch_ref, cw_ref,
                   cb_ref, out_ref):
    t = t2_ref[0] + t2_ref[1]
    g2 = jnp.maximum(dinv_ref[...] * (t + hn2_ref[...]) + b2_ref[...], 0.0)
    gids = lax.broadcasted_iota(jnp.int32, (NUM_GRAPHS, N), 0)
    onehot = (batch_ref[...] == gids).astype(jnp.float32)
    sums = jnp.dot(onehot, g2, **_DOT)
    counts = jnp.sum(onehot, axis=1, keepdims=True)
    pooled = sums / jnp.maximum(counts, 1.0)
    out_ref[...] = jnp.dot(pooled, cw_ref[...], **_DOT) + cb_ref[...]


def _tc_final(t2, hn2, dinv, b2row, batch_row, cls_W, cb_row):
    return pl.pallas_call(
        _tc_final_body,
        in_specs=[pl.BlockSpec((SC_CORES, N, HID), lambda: (0, 0, 0)),
                  pl.BlockSpec((N, HID), lambda: (0, 0)),
                  pl.BlockSpec((N, HID), lambda: (0, 0)),
                  pl.BlockSpec((1, HID), lambda: (0, 0)),
                  pl.BlockSpec((1, N), lambda: (0, 0)),
                  pl.BlockSpec((HID, NUM_CLASSES), lambda: (0, 0)),
                  pl.BlockSpec((1, NUM_CLASSES), lambda: (0, 0))],
        out_specs=pl.BlockSpec((NUM_GRAPHS, NUM_CLASSES), lambda: (0, 0)),
        out_shape=jax.ShapeDtypeStruct((NUM_GRAPHS, NUM_CLASSES), jnp.float32),
    )(t2, hn2, dinv, b2row, batch_row, cls_W, cb_row)


def kernel(x, edge_index, batch, pos, filt_W, filt_b, mlp_W1, mlp_b1, mlp_W2,
           mlp_b2, gcn1_W, gcn1_b, gcn2_W, gcn2_b, cls_W, cls_b):
    f32 = jnp.float32
    row = edge_index[0]
    col = edge_index[1]

    w1t, w1b = mlp_W1[:HID], mlp_W1[HID:]
    wa = jnp.einsum("cih,hj->icj", filt_W, w1t, **_DOT).reshape(3, NC * HID)
    wb = jnp.einsum("cih,hj->icj", filt_W, w1b, **_DOT).reshape(3, NC * HID)
    b1 = (jnp.einsum("ch,hj->cj", filt_b, w1t + w1b, **_DOT)
          + mlp_b1[None, :]).reshape(NC * HID)
    w24 = jnp.zeros((16, NC * HID), f32)
    w24 = w24.at[0:3].set(255.0 * wa)
    w24 = w24.at[7].set(b1)
    w24 = w24.at[8:11].set(255.0 * wb)
    thr = np.linspace(0.5, 0.01, NC)
    thr160_row = jnp.asarray(np.repeat(thr, HID)[None, :], f32)
    thr16_row = jnp.asarray(
        np.concatenate([thr, [np.inf] * 6])[None, :], f32)
    g3 = gcn1_W[3:].reshape(NC, HID, HID)
    wrelu = jnp.einsum("kj,cjh->ckh", mlp_W2, g3, **_DOT).reshape(NC * HID, HID)
    wcnt = jnp.einsum("j,cjh->ch", mlp_b2, g3, **_DOT)
    wc = jnp.concatenate([wrelu, wcnt, jnp.zeros((6, HID), f32)], axis=0)

    xs = x * (1.0 / 255.0)
    ps = pos * (1.0 / 32.0)
    p_tab = jnp.concatenate(
        [xs, ps, jnp.sum(xs * xs, axis=1, keepdims=True),
         jnp.sum(ps * ps, axis=1, keepdims=True), jnp.ones((N, 1), f32)],
        axis=1)
    zeros_w = jnp.zeros((N, SW2), f32)
    zeros_h = jnp.zeros((N, HID), f32)

    gr, gc = _sc_gather_records(p_tab, row, col)
    m = _tc_edge(gr, gc, w24, thr160_row, thr16_row, wc)
    s = _sc_scatter_m(m, col, zeros_w)
    hn, dinv = _tc_gcn1(x, s, gcn1_W[:3])
    t2 = _sc_gather_scatter(hn, row, col, zeros_h)
    hn2 = _tc_gcn2(t2, hn, dinv, gcn1_b[None, :], gcn2_W)
    t2b = _sc_gather_scatter(hn2, row, col, zeros_h)
    return _tc_final(t2b, hn2, dinv, gcn2_b[None, :], batch[None, :],
                     cls_W, cls_b[None, :])


# speedup vs baseline: 1.7092x; 1.7092x over previous
import functools

import jax
import jax.numpy as jnp
import numpy as np
from jax import lax
from jax.experimental import pallas as pl
from jax.experimental.pallas import tpu as pltpu
from jax.experimental.pallas import tpu_sc as plsc

N = 4096
E = 65536
HID = 16
NC = 10
NUM_GRAPHS = 16
NUM_CLASSES = 10
SC_CORES = 2
SC_SUBCORES = 16
NWORK = SC_CORES * SC_SUBCORES
EPW = E // NWORK
CH = 128
NCHUNK = EPW // CH
ROWS_PT = N // SC_SUBCORES

_mesh = plsc.VectorSubcoreMesh(
    core_axis_name="c", subcore_axis_name="s", num_cores=SC_CORES)
_SC_PARAMS = pltpu.CompilerParams(use_tc_tiling_on_sc=False)


@functools.partial(
    pl.kernel,
    out_type=(jax.ShapeDtypeStruct((E, 8), jnp.float32),
              jax.ShapeDtypeStruct((E, 8), jnp.float32)),
    mesh=_mesh,
    compiler_params=_SC_PARAMS,
    scratch_types=(pltpu.VMEM((CH,), jnp.int32),
                   pltpu.VMEM((CH,), jnp.int32),
                   pltpu.VMEM((CH, 8), jnp.float32),
                   pltpu.VMEM((CH, 8), jnp.float32),
                   pltpu.SemaphoreType.DMA,
                   pltpu.SemaphoreType.DMA),
)
def _sc_gather_records(p_hbm, row_hbm, col_hbm, gr_hbm, gc_hbm,
                       idxr_v, idxc_v, rr_v, rc_v, sem1, sem2):
    wid = lax.axis_index("s") * SC_CORES + lax.axis_index("c")
    base = wid * EPW

    @pl.loop(0, NCHUNK)
    def _chunk(i):
        off = base + i * CH
        pltpu.sync_copy(row_hbm.at[pl.ds(off, CH)], idxr_v)
        pltpu.sync_copy(col_hbm.at[pl.ds(off, CH)], idxc_v)
        cp1 = pltpu.async_copy(p_hbm.at[idxr_v], rr_v, sem1)
        cp2 = pltpu.async_copy(p_hbm.at[idxc_v], rc_v, sem2)
        cp1.wait()
        cp2.wait()
        pltpu.sync_copy(rr_v, gr_hbm.at[pl.ds(off, CH)])
        pltpu.sync_copy(rc_v, gc_hbm.at[pl.ds(off, CH)])


SW2 = 32


@functools.partial(
    pl.kernel,
    out_type=jax.ShapeDtypeStruct((SC_CORES, N, SW2), jnp.float32),
    mesh=_mesh,
    compiler_params=_SC_PARAMS,
    scratch_types=(pltpu.VMEM((CH,), jnp.int32),
                   pltpu.VMEM((CH, SW2), jnp.float32),
                   pltpu.VMEM_SHARED((N, SW2), jnp.float32)),
)
def _sc_scatter_m(m_hbm, col_hbm, zeros_hbm, out_hbm, idx_v, m_v, acc_sh):
    cid = lax.axis_index("c")
    sid = lax.axis_index("s")
    pltpu.sync_copy(zeros_hbm.at[pl.ds(sid * ROWS_PT, ROWS_PT)],
                    acc_sh.at[pl.ds(sid * ROWS_PT, ROWS_PT)])
    plsc.subcore_barrier()
    base = cid * (E // SC_CORES) + sid * EPW
    @pl.loop(0, NCHUNK)
    def _chunk(i):
        off = base + i * CH
        pltpu.sync_copy(col_hbm.at[pl.ds(off, CH)], idx_v)
        pltpu.sync_copy(m_hbm.at[pl.ds(off, CH)], m_v)
        pltpu.sync_copy(m_v, acc_sh.at[idx_v], add=True)
    plsc.subcore_barrier()
    pltpu.sync_copy(acc_sh.at[pl.ds(sid * ROWS_PT, ROWS_PT)],
                    out_hbm.at[cid, pl.ds(sid * ROWS_PT, ROWS_PT)])


@functools.partial(
    pl.kernel,
    out_type=jax.ShapeDtypeStruct((SC_CORES, N, HID), jnp.float32),
    mesh=_mesh,
    compiler_params=_SC_PARAMS,
    scratch_types=(pltpu.VMEM((CH,), jnp.int32),
                   pltpu.VMEM((CH,), jnp.int32),
                   pltpu.VMEM((CH, HID), jnp.float32),
                   pltpu.VMEM_SHARED((N, HID), jnp.float32),
                   pltpu.SemaphoreType.DMA),
)
def _sc_gather_scatter(hn_hbm, row_hbm, col_hbm, zeros_hbm, out_hbm,
                       idxr_v, idxc_v, rows_v, acc_sh, sem):
    cid = lax.axis_index("c")
    sid = lax.axis_index("s")
    pltpu.sync_copy(zeros_hbm.at[pl.ds(sid * ROWS_PT, ROWS_PT)],
                    acc_sh.at[pl.ds(sid * ROWS_PT, ROWS_PT)])
    plsc.subcore_barrier()
    base = cid * (E // SC_CORES) + sid * EPW
    @pl.loop(0, NCHUNK)
    def _chunk(i):
        off = base + i * CH
        pltpu.sync_copy(row_hbm.at[pl.ds(off, CH)], idxr_v)
        pltpu.sync_copy(col_hbm.at[pl.ds(off, CH)], idxc_v)
        pltpu.async_copy(hn_hbm.at[idxr_v], rows_v, sem).wait()
        pltpu.sync_copy(rows_v, acc_sh.at[idxc_v], add=True)
    plsc.subcore_barrier()
    pltpu.sync_copy(acc_sh.at[pl.ds(sid * ROWS_PT, ROWS_PT)],
                    out_hbm.at[cid, pl.ds(sid * ROWS_PT, ROWS_PT)])


_DOT = dict(precision=lax.Precision.HIGHEST, preferred_element_type=jnp.float32)
_DOTH = dict(precision=lax.Precision.DEFAULT, preferred_element_type=jnp.float32)
BE = 2048


def _tc_edge_body(gr_ref, gc_ref, w24_ref, thr160_ref, thr16_ref, wc_ref,
                  m_ref):
    gr = gr_ref[...]
    gc = gc_ref[...]
    w16 = w24_ref[...]
    out = (jnp.dot(gr, w16[0:8], **_DOTH)
           + jnp.dot(gc, w16[8:16], **_DOTH))
    relu1 = jnp.maximum(out, 0.0)
    prod = gr * gc
    d2pix = (gr[:, 5:6] + gc[:, 5:6]
             - 2.0 * jnp.sum(prod[:, 0:3], axis=1, keepdims=True))
    d2pos = (gr[:, 6:7] + gc[:, 6:7]
             - 2.0 * jnp.sum(prod[:, 3:5], axis=1, keepdims=True))
    dpix = jnp.sqrt(jnp.maximum(d2pix, 0.0))
    dpos = jnp.sqrt(jnp.maximum(d2pos, 0.0))
    ev = (jnp.exp(dpos * (-1.0 / ((0.05 * np.pi) ** 2)))
          + jnp.exp(dpix * (-1.0 / (0.05 ** 2))))
    m160 = jnp.where(ev >= thr160_ref[...], relu1, 0.0)
    mask16 = jnp.where(ev >= thr16_ref[...], 1.0, 0.0)
    wc = wc_ref[...]
    v = (jnp.dot(m160, wc[0:160], **_DOTH)
         + jnp.dot(mask16, wc[160:176], **_DOTH))
    m_ref[...] = jnp.concatenate(
        [v, jnp.ones((BE, 1), jnp.float32), jnp.zeros((BE, 15), jnp.float32)],
        axis=1)


def _tc_edge(gr, gc, w24, thr160_row, thr16_row, wc):
    grid = E // BE
    return pl.pallas_call(
        _tc_edge_body,
        grid=(grid,),
        in_specs=[
            pl.BlockSpec((BE, 8), lambda i: (i, 0)),
            pl.BlockSpec((BE, 8), lambda i: (i, 0)),
            pl.BlockSpec((16, NC * HID), lambda i: (0, 0)),
            pl.BlockSpec((1, NC * HID), lambda i: (0, 0)),
            pl.BlockSpec((1, 16), lambda i: (0, 0)),
            pl.BlockSpec((176, HID), lambda i: (0, 0)),
        ],
        out_specs=pl.BlockSpec((BE, SW2), lambda i: (i, 0)),
        out_shape=jax.ShapeDtypeStruct((E, SW2), jnp.float32),
        compiler_params=pltpu.CompilerParams(
            dimension_semantics=("arbitrary",)),
    )(gr, gc, w24, thr160_row, thr16_row, wc)


def _tc_gcn1_body(x_ref, s_ref, wx_ref, hn_ref, dinv_ref):
    s = s_ref[0] + s_ref[1]
    deg = 1.0 + s[:, HID:HID + 1]
    dinv = lax.rsqrt(deg)
    h0 = jnp.dot(x_ref[...], wx_ref[...], **_DOT) + s[:, 0:HID]
    hn_ref[...] = h0 * dinv
    dinv_ref[...] = jnp.broadcast_to(dinv, (N, HID))


def _tc_gcn1(x, s, wx3):
    return pl.pallas_call(
        _tc_gcn1_body,
        in_specs=[pl.BlockSpec((N, 3), lambda: (0, 0)),
                  pl.BlockSpec((SC_CORES, N, SW2), lambda: (0, 0, 0)),
                  pl.BlockSpec((3, HID), lambda: (0, 0))],
        out_specs=[pl.BlockSpec((N, HID), lambda: (0, 0)),
                   pl.BlockSpec((N, HID), lambda: (0, 0))],
        out_shape=[jax.ShapeDtypeStruct((N, HID), jnp.float32),
                   jax.ShapeDtypeStruct((N, HID), jnp.float32)],
    )(x, s, wx3)


def _tc_gcn2_body(t2_ref, hn_ref, dinv_ref, b1_ref, w2_ref, hn2_ref):
    t = t2_ref[0] + t2_ref[1]
    g1 = jnp.maximum(dinv_ref[...] * (t + hn_ref[...]) + b1_ref[...], 0.0)
    hn2_ref[...] = jnp.dot(g1, w2_ref[...], **_DOT) * dinv_ref[...]


def _tc_gcn2(t2, hn, dinv, b1row, w2):
    return pl.pallas_call(
        _tc_gcn2_body,
        in_specs=[pl.BlockSpec((SC_CORES, N, HID), lambda: (0, 0, 0)),
                  pl.BlockSpec((N, HID), lambda: (0, 0)),
                  pl.BlockSpec((N, HID), lambda: (0, 0)),
                  pl.BlockSpec((1, HID), lambda: (0, 0)),
                  pl.BlockSpec((HID, HID), lambda: (0, 0))],
        out_specs=pl.BlockSpec((N, HID), lambda: (0, 0)),
        out_shape=jax.ShapeDtypeStruct((N, HID), jnp.float32),
    )(t2, hn, dinv, b1row, w2)


def _tc_final_body(t2_ref, hn2_ref, dinv_ref, b2_ref, batch_ref, cw_ref,
                   cb_ref, out_ref):
    t = t2_ref[0] + t2_ref[1]
    g2 = jnp.maximum(dinv_ref[...] * (t + hn2_ref[...]) + b2_ref[...], 0.0)
    gids = lax.broadcasted_iota(jnp.int32, (NUM_GRAPHS, N), 0)
    onehot = (batch_ref[...] == gids).astype(jnp.float32)
    sums = jnp.dot(onehot, g2, **_DOT)
    counts = jnp.sum(onehot, axis=1, keepdims=True)
    pooled = sums / jnp.maximum(counts, 1.0)
    out_ref[...] = jnp.dot(pooled, cw_ref[...], **_DOT) + cb_ref[...]


def _tc_final(t2, hn2, dinv, b2row, batch_row, cls_W, cb_row):
    return pl.pallas_call(
        _tc_final_body,
        in_specs=[pl.BlockSpec((SC_CORES, N, HID), lambda: (0, 0, 0)),
                  pl.BlockSpec((N, HID), lambda: (0, 0)),
                  pl.BlockSpec((N, HID), lambda: (0, 0)),
                  pl.BlockSpec((1, HID), lambda: (0, 0)),
                  pl.BlockSpec((1, N), lambda: (0, 0)),
                  pl.BlockSpec((HID, NUM_CLASSES), lambda: (0, 0)),
                  pl.BlockSpec((1, NUM_CLASSES), lambda: (0, 0))],
        out_specs=pl.BlockSpec((NUM_GRAPHS, NUM_CLASSES), lambda: (0, 0)),
        out_shape=jax.ShapeDtypeStruct((NUM_GRAPHS, NUM_CLASSES), jnp.float32),
    )(t2, hn2, dinv, b2row, batch_row, cls_W, cb_row)


def kernel(x, edge_index, batch, pos, filt_W, filt_b, mlp_W1, mlp_b1, mlp_W2,
           mlp_b2, gcn1_W, gcn1_b, gcn2_W, gcn2_b, cls_W, cls_b):
    f32 = jnp.float32
    row = edge_index[0]
    col = edge_index[1]

    w1t, w1b = mlp_W1[:HID], mlp_W1[HID:]
    wa = jnp.einsum("cih,hj->icj", filt_W, w1t, **_DOT).reshape(3, NC * HID)
    wb = jnp.einsum("cih,hj->icj", filt_W, w1b, **_DOT).reshape(3, NC * HID)
    b1 = (jnp.einsum("ch,hj->cj", filt_b, w1t + w1b, **_DOT)
          + mlp_b1[None, :]).reshape(NC * HID)
    w24 = jnp.zeros((16, NC * HID), f32)
    w24 = w24.at[0:3].set(255.0 * wa)
    w24 = w24.at[7].set(b1)
    w24 = w24.at[8:11].set(255.0 * wb)
    thr = np.linspace(0.5, 0.01, NC)
    thr160_row = jnp.asarray(np.repeat(thr, HID)[None, :], f32)
    thr16_row = jnp.asarray(
        np.concatenate([thr, [np.inf] * 6])[None, :], f32)
    g3 = gcn1_W[3:].reshape(NC, HID, HID)
    wrelu = jnp.einsum("kj,cjh->ckh", mlp_W2, g3, **_DOT).reshape(NC * HID, HID)
    wcnt = jnp.einsum("j,cjh->ch", mlp_b2, g3, **_DOT)
    wc = jnp.concatenate([wrelu, wcnt, jnp.zeros((6, HID), f32)], axis=0)

    xs = x * (1.0 / 255.0)
    ps = pos * (1.0 / 32.0)
    p_tab = jnp.concatenate(
        [xs, ps, jnp.sum(xs * xs, axis=1, keepdims=True),
         jnp.sum(ps * ps, axis=1, keepdims=True), jnp.ones((N, 1), f32)],
        axis=1)
    zeros_w = jnp.zeros((N, SW2), f32)
    zeros_h = jnp.zeros((N, HID), f32)

    gr, gc = _sc_gather_records(p_tab, row, col)
    m = _tc_edge(gr, gc, w24, thr160_row, thr16_row, wc)
    s = _sc_scatter_m(m, col, zeros_w)
    hn, dinv = _tc_gcn1(x, s, gcn1_W[:3])
    t2 = _sc_gather_scatter(hn, row, col, zeros_h)
    hn2 = _tc_gcn2(t2, hn, dinv, gcn1_b[None, :], gcn2_W)
    t2b = _sc_gather_scatter(hn2, row, col, zeros_h)
    return _tc_final(t2b, hn2, dinv, gcn2_b[None, :], batch[None, :],
                     cls_W, cls_b[None, :])


# speedup vs baseline: 1.9666x; 1.1506x over previous
import functools

import jax
import jax.numpy as jnp
import numpy as np
from jax import lax
from jax.experimental import pallas as pl
from jax.experimental.pallas import tpu as pltpu
from jax.experimental.pallas import tpu_sc as plsc

N = 4096
E = 65536
HID = 16
NC = 10
NUM_GRAPHS = 16
NUM_CLASSES = 10
SC_CORES = 2
SC_SUBCORES = 16
NWORK = SC_CORES * SC_SUBCORES
EPW = E // NWORK
CH = 128
NCHUNK = EPW // CH
ROWS_PT = N // SC_SUBCORES

_mesh = plsc.VectorSubcoreMesh(
    core_axis_name="c", subcore_axis_name="s", num_cores=SC_CORES)
_SC_PARAMS = pltpu.CompilerParams(use_tc_tiling_on_sc=False)


@functools.partial(
    pl.kernel,
    out_type=(jax.ShapeDtypeStruct((E, 8), jnp.float32),
              jax.ShapeDtypeStruct((E, 8), jnp.float32)),
    mesh=_mesh,
    compiler_params=_SC_PARAMS,
    scratch_types=(pltpu.VMEM((CH,), jnp.int32),
                   pltpu.VMEM((CH,), jnp.int32),
                   pltpu.VMEM((CH, 8), jnp.float32),
                   pltpu.VMEM((CH, 8), jnp.float32),
                   pltpu.SemaphoreType.DMA,
                   pltpu.SemaphoreType.DMA),
)
def _sc_gather_records(p_hbm, row_hbm, col_hbm, gr_hbm, gc_hbm,
                       idxr_v, idxc_v, rr_v, rc_v, sem1, sem2):
    wid = lax.axis_index("s") * SC_CORES + lax.axis_index("c")
    base = wid * EPW

    @pl.loop(0, NCHUNK)
    def _chunk(i):
        off = base + i * CH
        pltpu.sync_copy(row_hbm.at[pl.ds(off, CH)], idxr_v)
        pltpu.sync_copy(col_hbm.at[pl.ds(off, CH)], idxc_v)
        cp1 = pltpu.async_copy(p_hbm.at[idxr_v], rr_v, sem1)
        cp2 = pltpu.async_copy(p_hbm.at[idxc_v], rc_v, sem2)
        cp1.wait()
        cp2.wait()
        pltpu.sync_copy(rr_v, gr_hbm.at[pl.ds(off, CH)])
        pltpu.sync_copy(rc_v, gc_hbm.at[pl.ds(off, CH)])


SW2 = 32


@functools.partial(
    pl.kernel,
    out_type=jax.ShapeDtypeStruct((SC_CORES, N, SW2), jnp.float32),
    mesh=_mesh,
    compiler_params=_SC_PARAMS,
    scratch_types=(pltpu.VMEM((CH,), jnp.int32),
                   pltpu.VMEM((CH, SW2), jnp.float32),
                   pltpu.VMEM_SHARED((N, SW2), jnp.float32)),
)
def _sc_scatter_m(m_hbm, col_hbm, zeros_hbm, out_hbm, idx_v, m_v, acc_sh):
    cid = lax.axis_index("c")
    sid = lax.axis_index("s")
    pltpu.sync_copy(zeros_hbm.at[pl.ds(sid * ROWS_PT, ROWS_PT)],
                    acc_sh.at[pl.ds(sid * ROWS_PT, ROWS_PT)])
    plsc.subcore_barrier()
    base = cid * (E // SC_CORES) + sid * EPW
    @pl.loop(0, NCHUNK)
    def _chunk(i):
        off = base + i * CH
        pltpu.sync_copy(col_hbm.at[pl.ds(off, CH)], idx_v)
        pltpu.sync_copy(m_hbm.at[pl.ds(off, CH)], m_v)
        pltpu.sync_copy(m_v, acc_sh.at[idx_v], add=True)
    plsc.subcore_barrier()
    pltpu.sync_copy(acc_sh.at[pl.ds(sid * ROWS_PT, ROWS_PT)],
                    out_hbm.at[cid, pl.ds(sid * ROWS_PT, ROWS_PT)])


@functools.partial(
    pl.kernel,
    out_type=jax.ShapeDtypeStruct((SC_CORES, N, HID), jnp.float32),
    mesh=_mesh,
    compiler_params=_SC_PARAMS,
    scratch_types=(pltpu.VMEM((CH,), jnp.int32),
                   pltpu.VMEM((CH,), jnp.int32),
                   pltpu.VMEM((CH, HID), jnp.float32),
                   pltpu.VMEM_SHARED((N, HID), jnp.float32),
                   pltpu.SemaphoreType.DMA),
)
def _sc_gather_scatter(hn_hbm, row_hbm, col_hbm, zeros_hbm, out_hbm,
                       idxr_v, idxc_v, rows_v, acc_sh, sem):
    cid = lax.axis_index("c")
    sid = lax.axis_index("s")
    pltpu.sync_copy(zeros_hbm.at[pl.ds(sid * ROWS_PT, ROWS_PT)],
                    acc_sh.at[pl.ds(sid * ROWS_PT, ROWS_PT)])
    plsc.subcore_barrier()
    base = cid * (E // SC_CORES) + sid * EPW
    @pl.loop(0, NCHUNK)
    def _chunk(i):
        off = base + i * CH
        pltpu.sync_copy(row_hbm.at[pl.ds(off, CH)], idxr_v)
        pltpu.sync_copy(col_hbm.at[pl.ds(off, CH)], idxc_v)
        pltpu.async_copy(hn_hbm.at[idxr_v], rows_v, sem).wait()
        pltpu.sync_copy(rows_v, acc_sh.at[idxc_v], add=True)
    plsc.subcore_barrier()
    pltpu.sync_copy(acc_sh.at[pl.ds(sid * ROWS_PT, ROWS_PT)],
                    out_hbm.at[cid, pl.ds(sid * ROWS_PT, ROWS_PT)])


_DOT = dict(precision=lax.Precision.HIGHEST, preferred_element_type=jnp.float32)
_DOTH = dict(precision=lax.Precision.DEFAULT, preferred_element_type=jnp.float32)
BE = 8192


def _tc_edge_body(gr_ref, gc_ref, w24_ref, thr160_ref, thr16_ref, wc_ref,
                  m_ref):
    gr = gr_ref[...]
    gc = gc_ref[...]
    w16 = w24_ref[...]
    out = (jnp.dot(gr, w16[0:8], **_DOTH)
           + jnp.dot(gc, w16[8:16], **_DOTH))
    relu1 = jnp.maximum(out, 0.0)
    prod = gr * gc
    d2pix = (gr[:, 5:6] + gc[:, 5:6]
             - 2.0 * jnp.sum(prod[:, 0:3], axis=1, keepdims=True))
    d2pos = (gr[:, 6:7] + gc[:, 6:7]
             - 2.0 * jnp.sum(prod[:, 3:5], axis=1, keepdims=True))
    dpix = jnp.sqrt(jnp.maximum(d2pix, 0.0))
    dpos = jnp.sqrt(jnp.maximum(d2pos, 0.0))
    ev = (jnp.exp(dpos * (-1.0 / ((0.05 * np.pi) ** 2)))
          + jnp.exp(dpix * (-1.0 / (0.05 ** 2))))
    m160 = jnp.where(ev >= thr160_ref[...], relu1, 0.0)
    mask16 = jnp.where(ev >= thr16_ref[...], 1.0, 0.0)
    wc = wc_ref[...]
    v = (jnp.dot(m160, wc[0:160], **_DOTH)
         + jnp.dot(mask16, wc[160:176], **_DOTH))
    m_ref[...] = jnp.concatenate(
        [v, jnp.ones((BE, 1), jnp.float32), jnp.zeros((BE, 15), jnp.float32)],
        axis=1)


def _tc_edge(gr, gc, w24, thr160_row, thr16_row, wc):
    grid = E // BE
    return pl.pallas_call(
        _tc_edge_body,
        grid=(grid,),
        in_specs=[
            pl.BlockSpec((BE, 8), lambda i: (i, 0)),
            pl.BlockSpec((BE, 8), lambda i: (i, 0)),
            pl.BlockSpec((16, NC * HID), lambda i: (0, 0)),
            pl.BlockSpec((1, NC * HID), lambda i: (0, 0)),
            pl.BlockSpec((1, 16), lambda i: (0, 0)),
            pl.BlockSpec((176, HID), lambda i: (0, 0)),
        ],
        out_specs=pl.BlockSpec((BE, SW2), lambda i: (i, 0)),
        out_shape=jax.ShapeDtypeStruct((E, SW2), jnp.float32),
        compiler_params=pltpu.CompilerParams(
            dimension_semantics=("arbitrary",)),
    )(gr, gc, w24, thr160_row, thr16_row, wc)


def _tc_gcn1_body(x_ref, s_ref, wx_ref, hn_ref, dinv_ref):
    s = s_ref[0] + s_ref[1]
    deg = 1.0 + s[:, HID:HID + 1]
    dinv = lax.rsqrt(deg)
    h0 = jnp.dot(x_ref[...], wx_ref[...], **_DOT) + s[:, 0:HID]
    hn_ref[...] = h0 * dinv
    dinv_ref[...] = jnp.broadcast_to(dinv, (N, HID))


def _tc_gcn1(x, s, wx3):
    return pl.pallas_call(
        _tc_gcn1_body,
        in_specs=[pl.BlockSpec((N, 3), lambda: (0, 0)),
                  pl.BlockSpec((SC_CORES, N, SW2), lambda: (0, 0, 0)),
                  pl.BlockSpec((3, HID), lambda: (0, 0))],
        out_specs=[pl.BlockSpec((N, HID), lambda: (0, 0)),
                   pl.BlockSpec((N, HID), lambda: (0, 0))],
        out_shape=[jax.ShapeDtypeStruct((N, HID), jnp.float32),
                   jax.ShapeDtypeStruct((N, HID), jnp.float32)],
    )(x, s, wx3)


def _tc_gcn2_body(t2_ref, hn_ref, dinv_ref, b1_ref, w2_ref, hn2_ref):
    t = t2_ref[0] + t2_ref[1]
    g1 = jnp.maximum(dinv_ref[...] * (t + hn_ref[...]) + b1_ref[...], 0.0)
    hn2_ref[...] = jnp.dot(g1, w2_ref[...], **_DOT) * dinv_ref[...]


def _tc_gcn2(t2, hn, dinv, b1row, w2):
    return pl.pallas_call(
        _tc_gcn2_body,
        in_specs=[pl.BlockSpec((SC_CORES, N, HID), lambda: (0, 0, 0)),
                  pl.BlockSpec((N, HID), lambda: (0, 0)),
                  pl.BlockSpec((N, HID), lambda: (0, 0)),
                  pl.BlockSpec((1, HID), lambda: (0, 0)),
                  pl.BlockSpec((HID, HID), lambda: (0, 0))],
        out_specs=pl.BlockSpec((N, HID), lambda: (0, 0)),
        out_shape=jax.ShapeDtypeStruct((N, HID), jnp.float32),
    )(t2, hn, dinv, b1row, w2)


def _tc_final_body(t2_ref, hn2_ref, dinv_ref, b2_ref, batch_ref, cw_ref,
                   cb_ref, out_ref):
    t = t2_ref[0] + t2_ref[1]
    g2 = jnp.maximum(dinv_ref[...] * (t + hn2_ref[...]) + b2_ref[...], 0.0)
    gids = lax.broadcasted_iota(jnp.int32, (NUM_GRAPHS, N), 0)
    onehot = (batch_ref[...] == gids).astype(jnp.float32)
    sums = jnp.dot(onehot, g2, **_DOT)
    counts = jnp.sum(onehot, axis=1, keepdims=True)
    pooled = sums / jnp.maximum(counts, 1.0)
    out_ref[...] = jnp.dot(pooled, cw_ref[...], **_DOT) + cb_ref[...]


def _tc_final(t2, hn2, dinv, b2row, batch_row, cls_W, cb_row):
    return pl.pallas_call(
        _tc_final_body,
        in_specs=[pl.BlockSpec((SC_CORES, N, HID), lambda: (0, 0, 0)),
                  pl.BlockSpec((N, HID), lambda: (0, 0)),
                  pl.BlockSpec((N, HID), lambda: (0, 0)),
                  pl.BlockSpec((1, HID), lambda: (0, 0)),
                  pl.BlockSpec((1, N), lambda: (0, 0)),
                  pl.BlockSpec((HID, NUM_CLASSES), lambda: (0, 0)),
                  pl.BlockSpec((1, NUM_CLASSES), lambda: (0, 0))],
        out_specs=pl.BlockSpec((NUM_GRAPHS, NUM_CLASSES), lambda: (0, 0)),
        out_shape=jax.ShapeDtypeStruct((NUM_GRAPHS, NUM_CLASSES), jnp.float32),
    )(t2, hn2, dinv, b2row, batch_row, cls_W, cb_row)


def kernel(x, edge_index, batch, pos, filt_W, filt_b, mlp_W1, mlp_b1, mlp_W2,
           mlp_b2, gcn1_W, gcn1_b, gcn2_W, gcn2_b, cls_W, cls_b):
    f32 = jnp.float32
    row = edge_index[0]
    col = edge_index[1]

    w1t, w1b = mlp_W1[:HID], mlp_W1[HID:]
    wa = jnp.einsum("cih,hj->icj", filt_W, w1t, **_DOT).reshape(3, NC * HID)
    wb = jnp.einsum("cih,hj->icj", filt_W, w1b, **_DOT).reshape(3, NC * HID)
    b1 = (jnp.einsum("ch,hj->cj", filt_b, w1t + w1b, **_DOT)
          + mlp_b1[None, :]).reshape(NC * HID)
    w24 = jnp.zeros((16, NC * HID), f32)
    w24 = w24.at[0:3].set(255.0 * wa)
    w24 = w24.at[7].set(b1)
    w24 = w24.at[8:11].set(255.0 * wb)
    thr = np.linspace(0.5, 0.01, NC)
    thr160_row = jnp.asarray(np.repeat(thr, HID)[None, :], f32)
    thr16_row = jnp.asarray(
        np.concatenate([thr, [np.inf] * 6])[None, :], f32)
    g3 = gcn1_W[3:].reshape(NC, HID, HID)
    wrelu = jnp.einsum("kj,cjh->ckh", mlp_W2, g3, **_DOT).reshape(NC * HID, HID)
    wcnt = jnp.einsum("j,cjh->ch", mlp_b2, g3, **_DOT)
    wc = jnp.concatenate([wrelu, wcnt, jnp.zeros((6, HID), f32)], axis=0)

    xs = x * (1.0 / 255.0)
    ps = pos * (1.0 / 32.0)
    p_tab = jnp.concatenate(
        [xs, ps, jnp.sum(xs * xs, axis=1, keepdims=True),
         jnp.sum(ps * ps, axis=1, keepdims=True), jnp.ones((N, 1), f32)],
        axis=1)
    zeros_w = jnp.zeros((N, SW2), f32)
    zeros_h = jnp.zeros((N, HID), f32)

    gr, gc = _sc_gather_records(p_tab, row, col)
    m = _tc_edge(gr, gc, w24, thr160_row, thr16_row, wc)
    s = _sc_scatter_m(m, col, zeros_w)
    hn, dinv = _tc_gcn1(x, s, gcn1_W[:3])
    t2 = _sc_gather_scatter(hn, row, col, zeros_h)
    hn2 = _tc_gcn2(t2, hn, dinv, gcn1_b[None, :], gcn2_W)
    t2b = _sc_gather_scatter(hn2, row, col, zeros_h)
    return _tc_final(t2b, hn2, dinv, gcn2_b[None, :], batch[None, :],
                     cls_W, cls_b[None, :])
